# baseline (device time: 146878 ns/iter reference)
import jax
import jax.numpy as jnp
from jax import lax
from jax.experimental import pallas as pl
from jax.experimental.pallas import tpu as pltpu

N_DEV = 4
SQ = 2048
SKV = 2048
D_MODEL = 1024
H_LOC = 8
DH = 128
BLK = 64
SCALE = 0.08838834764831843
QT = 512
N_QT = SQ // QT
CHUNK = SQ // (2 * N_DEV)


def _compute_body(
    x_ref, wq_hbm, k_ref, v_ref, wo_hbm, out_ref,
    wq_s, wo_s, xb, wqb, wob, load_sems,
):
    h = pl.program_id(0)
    my = lax.axis_index("i")

    @pl.when(h == 0)
    def _():
        cpq = pltpu.make_async_copy(
            wq_hbm.at[:, pl.ds(my * (H_LOC * DH), H_LOC * DH)],
            wq_s, load_sems.at[0],
        )
        cpo = pltpu.make_async_copy(
            wo_hbm.at[pl.ds(my * (H_LOC * DH), H_LOC * DH), :],
            wo_s, load_sems.at[1],
        )
        cpq.start()
        cpo.start()
        xb[...] = x_ref[...].astype(jnp.bfloat16)
        cpq.wait()
        cpo.wait()
        wqb[...] = wq_s[...].astype(jnp.bfloat16)
        wob[...] = wo_s[...].astype(jnp.bfloat16)

    q = jnp.dot(
        xb[...], wqb[:, pl.ds(h * DH, DH)],
        preferred_element_type=jnp.float32,
    )
    qb = (q * SCALE).astype(jnp.bfloat16)
    k = k_ref[...].astype(jnp.bfloat16)
    v = v_ref[...].astype(jnp.bfloat16)
    ctxs = []
    for qt in range(N_QT):
        qq = qb[qt * QT:(qt + 1) * QT, :]
        sd = lax.dot_general(
            qq, k[qt * QT:(qt + 1) * QT, :], (((1,), (1,)), ((), ())),
            preferred_element_type=jnp.float32,
        )
        row = lax.broadcasted_iota(jnp.int32, (QT, QT), 0)
        col = lax.broadcasted_iota(jnp.int32, (QT, QT), 1)
        wd = jnp.where((col // BLK) <= (row // BLK), jnp.exp(sd), 0.0)
        denom = jnp.sum(wd, axis=-1, keepdims=True)
        ctx_u = jnp.dot(
            wd.astype(jnp.bfloat16), v[qt * QT:(qt + 1) * QT, :],
            preferred_element_type=jnp.float32,
        )
        if qt > 0:
            so = lax.dot_general(
                qq, k[:qt * QT, :], (((1,), (1,)), ((), ())),
                preferred_element_type=jnp.float32,
            )
            wo_ = jnp.exp(so)
            denom = denom + jnp.sum(wo_, axis=-1, keepdims=True)
            ctx_u = ctx_u + jnp.dot(
                wo_.astype(jnp.bfloat16), v[:qt * QT, :],
                preferred_element_type=jnp.float32,
            )
        ctxs.append(ctx_u / denom)
    ctx = jnp.concatenate(ctxs, axis=0)
    contrib = jnp.dot(
        ctx.astype(jnp.bfloat16), wob[pl.ds(h * DH, DH), :],
        preferred_element_type=jnp.float32,
    )

    @pl.when(h == 0)
    def _():
        out_ref[...] = contrib.astype(jnp.bfloat16)

    @pl.when(h != 0)
    def _():
        out_ref[...] += contrib.astype(jnp.bfloat16)


def _allreduce_body(p_ref, out_ref, comm_ref, p_sems, send_sems, recv_sems):
    my = lax.axis_index("i")
    left = lax.rem(my + (N_DEV - 1), N_DEV)
    right = lax.rem(my + 1, N_DEV)

    def mod4(v):
        return lax.rem(v + 4 * N_DEV, N_DEV)

    def rows_r(c):
        return c * CHUNK

    def rows_l(c):
        return N_DEV * CHUNK + c * CHUNK

    barrier_sem = pltpu.get_barrier_semaphore()
    for nbr in (left, right):
        pl.semaphore_signal(
            barrier_sem, inc=1,
            device_id=(nbr,), device_id_type=pl.DeviceIdType.MESH,
        )
    pl.semaphore_wait(barrier_sem, 2)

    p_copies = []
    for step in range(N_DEV):
        for start in (rows_r(mod4(my - step)), rows_l(mod4(my + step))):
            cp = pltpu.make_async_copy(
                p_ref.at[pl.ds(start, CHUNK), :],
                out_ref.at[pl.ds(start, CHUNK), :],
                p_sems.at[len(p_copies)],
            )
            cp.start()
            p_copies.append(cp)

    def copy(src_start, dst_start, dst_is_out, dev, sem_idx):
        dst = out_ref if dst_is_out else comm_ref
        return pltpu.make_async_remote_copy(
            src_ref=out_ref.at[pl.ds(src_start, CHUNK), :],
            dst_ref=dst.at[pl.ds(dst_start, CHUNK), :],
            send_sem=send_sems.at[sem_idx],
            recv_sem=recv_sems.at[sem_idx],
            device_id=(dev,),
            device_id_type=pl.DeviceIdType.MESH,
        )

    for s in range(N_DEV - 1):
        if s == 0:
            p_copies[0].wait()
            p_copies[1].wait()
        r_send = copy(rows_r(mod4(my - s)), s * CHUNK, False, right, s)
        l_send = copy(rows_l(mod4(my + s)), (3 + s) * CHUNK, False, left, 3 + s)
        r_send.start()
        l_send.start()
        p_copies[2 * s + 2].wait()
        p_copies[2 * s + 3].wait()
        r_send.wait()
        l_send.wait()
        rr = rows_r(mod4(my - s - 1))
        rl = rows_l(mod4(my + s + 1))
        out_ref[pl.ds(rr, CHUNK), :] += comm_ref[pl.ds(s * CHUNK, CHUNK), :]
        out_ref[pl.ds(rl, CHUNK), :] += comm_ref[pl.ds((3 + s) * CHUNK, CHUNK), :]

    for s in range(N_DEV - 1):
        cr = rows_r(mod4(my + 1 - s))
        cl = rows_l(mod4(my - 1 + s))
        r_send = copy(cr, cr, True, right, 6 + s)
        l_send = copy(cl, cl, True, left, 9 + s)
        r_send.start()
        l_send.start()
        r_send.wait()
        l_send.wait()


def kernel(x, Wq, K_ext, V_ext, Wo):
    i = lax.axis_index("i")
    x2 = x.reshape(SQ, D_MODEL)
    K = K_ext.reshape(SKV, H_LOC * DH)
    V = V_ext.reshape(SKV, H_LOC * DH)

    partial = pl.pallas_call(
        _compute_body,
        grid=(H_LOC,),
        in_specs=[
            pl.BlockSpec((SQ, D_MODEL), lambda h: (0, 0)),
            pl.BlockSpec(memory_space=pl.ANY),
            pl.BlockSpec((SKV, DH), lambda h: (0, h)),
            pl.BlockSpec((SKV, DH), lambda h: (0, h)),
            pl.BlockSpec(memory_space=pl.ANY),
        ],
        out_specs=pl.BlockSpec((SQ, D_MODEL), lambda h: (0, 0)),
        out_shape=jax.ShapeDtypeStruct((SQ, D_MODEL), jnp.bfloat16),
        scratch_shapes=[
            pltpu.VMEM((D_MODEL, H_LOC * DH), jnp.float32),
            pltpu.VMEM((H_LOC * DH, D_MODEL), jnp.float32),
            pltpu.VMEM((SQ, D_MODEL), jnp.bfloat16),
            pltpu.VMEM((D_MODEL, H_LOC * DH), jnp.bfloat16),
            pltpu.VMEM((H_LOC * DH, D_MODEL), jnp.bfloat16),
            pltpu.SemaphoreType.DMA((2,)),
        ],
        compiler_params=pltpu.CompilerParams(
            vmem_limit_bytes=100 * 1024 * 1024
        ),
    )(x2, Wq, K, V, Wo)

    out = pl.pallas_call(
        _allreduce_body,
        out_shape=jax.ShapeDtypeStruct((SQ, D_MODEL), jnp.bfloat16),
        in_specs=[pl.BlockSpec(memory_space=pl.ANY)],
        out_specs=pl.BlockSpec(memory_space=pltpu.VMEM),
        scratch_shapes=[
            pltpu.VMEM((6 * CHUNK, D_MODEL), jnp.bfloat16),
            pltpu.SemaphoreType.DMA((8,)),
            pltpu.SemaphoreType.DMA((12,)),
            pltpu.SemaphoreType.DMA((12,)),
        ],
        compiler_params=pltpu.CompilerParams(
            collective_id=0, vmem_limit_bytes=100 * 1024 * 1024
        ),
    )(partial)

    return out.astype(jnp.float32).reshape(1, SQ, D_MODEL)


# device time: 146408 ns/iter; 1.0032x vs baseline; 1.0032x over previous
import jax
import jax.numpy as jnp
from jax import lax
from jax.experimental import pallas as pl
from jax.experimental.pallas import tpu as pltpu

N_DEV = 4
SQ = 2048
SKV = 2048
D_MODEL = 1024
H_LOC = 8
DH = 128
BLK = 64
SCALE = 0.08838834764831843
QT = 512
N_QT = SQ // QT
CHUNK = SQ // (2 * N_DEV)


def _compute_body(
    x_ref, wq_hbm, k_ref, v_ref, wo_hbm, out_ref, wq_s, wo_s, load_sems
):
    h = pl.program_id(0)
    my = lax.axis_index("i")

    @pl.when(h == 0)
    def _():
        cpq = pltpu.make_async_copy(
            wq_hbm.at[:, pl.ds(my * (H_LOC * DH), H_LOC * DH)],
            wq_s, load_sems.at[0],
        )
        cpo = pltpu.make_async_copy(
            wo_hbm.at[pl.ds(my * (H_LOC * DH), H_LOC * DH), :],
            wo_s, load_sems.at[1],
        )
        cpq.start()
        cpo.start()
        cpq.wait()
        cpo.wait()

    q = jnp.dot(
        x_ref[...], wq_s[:, pl.ds(h * DH, DH)],
        preferred_element_type=jnp.float32,
    )
    qb = (q * SCALE).astype(jnp.bfloat16)
    k = k_ref[...].astype(jnp.bfloat16)
    v = v_ref[...].astype(jnp.bfloat16)
    ctxs = []
    for qt in range(N_QT):
        qq = qb[qt * QT:(qt + 1) * QT, :]
        sd = lax.dot_general(
            qq, k[qt * QT:(qt + 1) * QT, :], (((1,), (1,)), ((), ())),
            preferred_element_type=jnp.float32,
        )
        row = lax.broadcasted_iota(jnp.int32, (QT, QT), 0)
        col = lax.broadcasted_iota(jnp.int32, (QT, QT), 1)
        wd = jnp.where(
            (col // BLK) <= (row // BLK), jnp.exp(sd), 0.0
        ).astype(jnp.bfloat16)
        denom = jnp.sum(wd, axis=-1, keepdims=True, dtype=jnp.float32)
        ctx_u = jnp.dot(
            wd, v[qt * QT:(qt + 1) * QT, :],
            preferred_element_type=jnp.float32,
        )
        if qt > 0:
            so = lax.dot_general(
                qq, k[:qt * QT, :], (((1,), (1,)), ((), ())),
                preferred_element_type=jnp.float32,
            )
            wo_ = jnp.exp(so).astype(jnp.bfloat16)
            denom = denom + jnp.sum(wo_, axis=-1, keepdims=True, dtype=jnp.float32)
            ctx_u = ctx_u + jnp.dot(
                wo_, v[:qt * QT, :],
                preferred_element_type=jnp.float32,
            )
        ctxs.append(ctx_u / denom)
    ctx = jnp.concatenate(ctxs, axis=0)
    contrib = jnp.dot(
        ctx, wo_s[pl.ds(h * DH, DH), :], preferred_element_type=jnp.float32
    )

    @pl.when(h == 0)
    def _():
        out_ref[...] = contrib.astype(jnp.bfloat16)

    @pl.when(h != 0)
    def _():
        out_ref[...] += contrib.astype(jnp.bfloat16)


def _allreduce_body(p_ref, out_ref, comm_ref, p_sems, send_sems, recv_sems):
    my = lax.axis_index("i")
    left = lax.rem(my + (N_DEV - 1), N_DEV)
    right = lax.rem(my + 1, N_DEV)

    def mod4(v):
        return lax.rem(v + 4 * N_DEV, N_DEV)

    def rows_r(c):
        return c * CHUNK

    def rows_l(c):
        return N_DEV * CHUNK + c * CHUNK

    barrier_sem = pltpu.get_barrier_semaphore()
    for nbr in (left, right):
        pl.semaphore_signal(
            barrier_sem, inc=1,
            device_id=(nbr,), device_id_type=pl.DeviceIdType.MESH,
        )
    pl.semaphore_wait(barrier_sem, 2)

    p_copies = []
    for step in range(N_DEV):
        for start in (rows_r(mod4(my - step)), rows_l(mod4(my + step))):
            cp = pltpu.make_async_copy(
                p_ref.at[pl.ds(start, CHUNK), :],
                out_ref.at[pl.ds(start, CHUNK), :],
                p_sems.at[len(p_copies)],
            )
            cp.start()
            p_copies.append(cp)

    def copy(src_start, dst_start, dst_is_out, dev, sem_idx):
        dst = out_ref if dst_is_out else comm_ref
        return pltpu.make_async_remote_copy(
            src_ref=out_ref.at[pl.ds(src_start, CHUNK), :],
            dst_ref=dst.at[pl.ds(dst_start, CHUNK), :],
            send_sem=send_sems.at[sem_idx],
            recv_sem=recv_sems.at[sem_idx],
            device_id=(dev,),
            device_id_type=pl.DeviceIdType.MESH,
        )

    for s in range(N_DEV - 1):
        if s == 0:
            p_copies[0].wait()
            p_copies[1].wait()
        r_send = copy(rows_r(mod4(my - s)), s * CHUNK, False, right, s)
        l_send = copy(rows_l(mod4(my + s)), (3 + s) * CHUNK, False, left, 3 + s)
        r_send.start()
        l_send.start()
        p_copies[2 * s + 2].wait()
        p_copies[2 * s + 3].wait()
        r_send.wait()
        l_send.wait()
        rr = rows_r(mod4(my - s - 1))
        rl = rows_l(mod4(my + s + 1))
        out_ref[pl.ds(rr, CHUNK), :] += comm_ref[pl.ds(s * CHUNK, CHUNK), :]
        out_ref[pl.ds(rl, CHUNK), :] += comm_ref[pl.ds((3 + s) * CHUNK, CHUNK), :]

    for s in range(N_DEV - 1):
        cr = rows_r(mod4(my + 1 - s))
        cl = rows_l(mod4(my - 1 + s))
        r_send = copy(cr, cr, True, right, 6 + s)
        l_send = copy(cl, cl, True, left, 9 + s)
        r_send.start()
        l_send.start()
        r_send.wait()
        l_send.wait()


def kernel(x, Wq, K_ext, V_ext, Wo):
    i = lax.axis_index("i")
    x2 = x.reshape(SQ, D_MODEL)
    K = K_ext.reshape(SKV, H_LOC * DH)
    V = V_ext.reshape(SKV, H_LOC * DH)

    partial = pl.pallas_call(
        _compute_body,
        grid=(H_LOC,),
        in_specs=[
            pl.BlockSpec((SQ, D_MODEL), lambda h: (0, 0)),
            pl.BlockSpec(memory_space=pl.ANY),
            pl.BlockSpec((SKV, DH), lambda h: (0, h)),
            pl.BlockSpec((SKV, DH), lambda h: (0, h)),
            pl.BlockSpec(memory_space=pl.ANY),
        ],
        out_specs=pl.BlockSpec((SQ, D_MODEL), lambda h: (0, 0)),
        out_shape=jax.ShapeDtypeStruct((SQ, D_MODEL), jnp.bfloat16),
        scratch_shapes=[
            pltpu.VMEM((D_MODEL, H_LOC * DH), jnp.float32),
            pltpu.VMEM((H_LOC * DH, D_MODEL), jnp.float32),
            pltpu.SemaphoreType.DMA((2,)),
        ],
        compiler_params=pltpu.CompilerParams(
            vmem_limit_bytes=100 * 1024 * 1024
        ),
    )(x2, Wq, K, V, Wo)

    out = pl.pallas_call(
        _allreduce_body,
        out_shape=jax.ShapeDtypeStruct((SQ, D_MODEL), jnp.bfloat16),
        in_specs=[pl.BlockSpec(memory_space=pl.ANY)],
        out_specs=pl.BlockSpec(memory_space=pltpu.VMEM),
        scratch_shapes=[
            pltpu.VMEM((6 * CHUNK, D_MODEL), jnp.bfloat16),
            pltpu.SemaphoreType.DMA((8,)),
            pltpu.SemaphoreType.DMA((12,)),
            pltpu.SemaphoreType.DMA((12,)),
        ],
        compiler_params=pltpu.CompilerParams(
            collective_id=0, vmem_limit_bytes=100 * 1024 * 1024
        ),
    )(partial)

    return out.astype(jnp.float32).reshape(1, SQ, D_MODEL)
